# baseline (speedup 1.0000x reference)
import functools

import jax
import jax.numpy as jnp
from jax import lax
from jax.experimental import pallas as pl
from jax.experimental.pallas import tpu as pltpu
from jax.experimental.pallas import tpu_sc as plsc

B = 4096
L = 20
D = 1000
LP = 32
ROW_W = D * L
NW = 32
RPW = B // NW
C = 2
NBUF = 2
CHUNK_W = C * ROW_W
GPW = RPW // C


def _sc_one_hot(x_pad_flat):
    mesh = plsc.VectorSubcoreMesh(core_axis_name="c", subcore_axis_name="s")

    @functools.partial(
        pl.kernel,
        mesh=mesh,
        compiler_params=pltpu.CompilerParams(needs_layout_passes=False),
        out_type=jax.ShapeDtypeStruct((B * ROW_W,), jnp.float32),
        scratch_types=[
            pltpu.VMEM((RPW * LP,), jnp.int32),
            pltpu.VMEM((CHUNK_W,), jnp.float32),
            pltpu.VMEM((CHUNK_W,), jnp.float32),
            pltpu.SemaphoreType.DMA,
            pltpu.SemaphoreType.DMA,
        ],
    )
    def one_hot_kernel(x_hbm, out_hbm, xt, buf0, buf1, sem0, sem1):
        wid = lax.axis_index("s") * 2 + lax.axis_index("c")
        obase = wid * (RPW * ROW_W)
        bufs = (buf0, buf1)
        sems = (sem0, sem1)

        lanes = lax.iota(jnp.int32, 16)
        tail_mask = lanes < (L - 16)
        ones_v = jnp.full((16,), 1.0, jnp.float32)
        zeros_v = jnp.zeros((16,), jnp.float32)

        pltpu.sync_copy(x_hbm.at[pl.ds(wid * (RPW * LP), RPW * LP)], xt)

        def zero_body(i, carry):
            buf0[pl.ds(i * 16, 16)] = zeros_v
            buf1[pl.ds(i * 16, 16)] = zeros_v
            return carry

        lax.fori_loop(0, CHUNK_W // 16, zero_body, 0)

        def scat(buf, g, val_v):
            for ri in range(C):
                base = (g * C + ri) * LP
                x1 = xt[pl.ds(base, 16)]
                x2 = xt[pl.ds(base + 16, 16)]
                idx1 = x1 * L + lanes + (ri * ROW_W)
                idx2 = x2 * L + (lanes + 16) + (ri * ROW_W)
                plsc.store_scatter(buf, [idx1], val_v)
                plsc.store_scatter(buf, [idx2], val_v, mask=tail_mask)

        def launch(b, g):
            pltpu.make_async_copy(
                bufs[b], out_hbm.at[pl.ds(obase + g * CHUNK_W, CHUNK_W)], sems[b]
            ).start()

        def drain(b, g):
            pltpu.make_async_copy(
                bufs[b], out_hbm.at[pl.ds(obase + g * CHUNK_W, CHUNK_W)], sems[b]
            ).wait()

        for b in range(NBUF):
            scat(bufs[b], b, ones_v)
            launch(b, b)

        def step(g0, carry):
            for b in range(NBUF):
                g = g0 * NBUF + b
                drain(b, g - NBUF)
                scat(bufs[b], g - NBUF, zeros_v)
                scat(bufs[b], g, ones_v)
                launch(b, g)
            return carry

        lax.fori_loop(1, GPW // NBUF, step, 0)

        for b in range(NBUF):
            drain(b, GPW - NBUF + b)

    return one_hot_kernel(x_pad_flat)


def kernel(X_in, ones):
    del ones
    x = jnp.pad(X_in.astype(jnp.int32), ((0, 0), (0, LP - L)))
    flat = _sc_one_hot(x.reshape(-1))
    return flat.reshape(B, D, L)


# speedup vs baseline: 3.0824x; 3.0824x over previous
import functools

import jax
import jax.numpy as jnp
from jax import lax
from jax.experimental import pallas as pl
from jax.experimental.pallas import tpu as pltpu
from jax.experimental.pallas import tpu_sc as plsc

B = 4096
L = 20
D = 1000
LP = 32
NW = 32
RPW = B // NW
CB = 2
DC = 200
ND = D // DC
NBUF = 2
TPW = (RPW // CB) * ND


def _sc_one_hot(x_pad_flat):
    mesh = plsc.VectorSubcoreMesh(core_axis_name="c", subcore_axis_name="s")

    @functools.partial(
        pl.kernel,
        mesh=mesh,
        compiler_params=pltpu.CompilerParams(needs_layout_passes=False),
        out_type=jax.ShapeDtypeStruct((B, D, L), jnp.float32),
        scratch_types=[
            pltpu.VMEM((RPW * LP,), jnp.int32),
            pltpu.VMEM((CB, DC, L), jnp.float32),
            pltpu.VMEM((CB, DC, L), jnp.float32),
            pltpu.SemaphoreType.DMA,
            pltpu.SemaphoreType.DMA,
        ],
    )
    def one_hot_kernel(x_hbm, out_hbm, xt, buf0, buf1, sem0, sem1):
        wid = lax.axis_index("s") * 2 + lax.axis_index("c")
        rbase = wid * RPW
        bufs = (buf0, buf1)
        sems = (sem0, sem1)

        lanes = lax.iota(jnp.int32, 16)
        tail_mask = lanes < (L - 16)
        ones_v = jnp.full((16,), 1.0, jnp.float32)
        zeros_v = jnp.zeros((16,), jnp.float32)

        pltpu.sync_copy(x_hbm.at[pl.ds(wid * (RPW * LP), RPW * LP)], xt)

        def zero_body(d, carry):
            for ri in range(CB):
                buf0[ri, d, pl.ds(0, 16)] = zeros_v
                buf0[ri, d, pl.ds(L - 16, 16)] = zeros_v
                buf1[ri, d, pl.ds(0, 16)] = zeros_v
                buf1[ri, d, pl.ds(L - 16, 16)] = zeros_v
            return carry

        lax.fori_loop(0, DC, zero_body, 0)

        def scat(buf, t, val_v):
            pb = t // ND
            d0 = (t % ND) * DC
            for ri in range(CB):
                base = (pb * CB + ri) * LP
                ri_v = jnp.full((16,), ri, jnp.int32)
                x1 = xt[pl.ds(base, 16)] - d0
                m1 = (x1 >= 0) & (x1 < DC)
                plsc.store_scatter(buf, [ri_v, x1, lanes], val_v, mask=m1)
                x2 = xt[pl.ds(base + 16, 16)] - d0
                m2 = (x2 >= 0) & (x2 < DC) & tail_mask
                plsc.store_scatter(buf, [ri_v, x2, lanes + 16], val_v, mask=m2)

        def dma(b, t):
            pb = t // ND
            d0 = (t % ND) * DC
            return pltpu.make_async_copy(
                bufs[b],
                out_hbm.at[pl.ds(rbase + pb * CB, CB), pl.ds(d0, DC), :],
                sems[b],
            )

        for b in range(NBUF):
            scat(bufs[b], b, ones_v)
            dma(b, b).start()

        def step(g0, carry):
            for b in range(NBUF):
                t = g0 * NBUF + b
                dma(b, t - NBUF).wait()
                scat(bufs[b], t - NBUF, zeros_v)
                scat(bufs[b], t, ones_v)
                dma(b, t).start()
            return carry

        lax.fori_loop(1, TPW // NBUF, step, 0)

        for b in range(NBUF):
            dma(b, TPW - NBUF + b).wait()

    return one_hot_kernel(x_pad_flat)


def kernel(X_in, ones):
    del ones
    x = jnp.pad(X_in.astype(jnp.int32), ((0, 0), (0, LP - L)))
    return _sc_one_hot(x.reshape(-1))
